# baseline (speedup 1.0000x reference)
import jax
import jax.numpy as jnp
from jax.experimental import pallas as pl

N = 10000
E = 320000
H = 256
N_STEPS = 4


def _combine_body(y_ref, f_ref, dt_ref, o_ref):
    o_ref[...] = y_ref[...] + dt_ref[0] * f_ref[...]


def _combine(y, f, dt):
    return pl.pallas_call(
        _combine_body,
        out_shape=jax.ShapeDtypeStruct(y.shape, y.dtype),
    )(y, f, dt)


def kernel(x, edge_index, W_emb, b_emb, Wq, bq, Wk, bk, Wv, bv, Ws, bs):
    src = edge_index[0]
    dst = edge_index[1]
    h = x @ W_emb + b_emb
    ts = jnp.linspace(0.0, 1.0, N_STEPS)
    ys = [h]
    y = h
    for i in range(N_STEPS - 1):
        t = ts[i]
        dt = ts[i + 1] - ts[i]
        tt = jnp.ones((N, 1), dtype=jnp.float32) * t
        inp = jnp.concatenate([tt, y], axis=1)
        q = inp @ Wq + bq
        k = inp @ Wk + bk
        v = inp @ Wv + bv
        score = jnp.sum(q[dst] * k[src], axis=-1) / jnp.sqrt(jnp.float32(H))
        m = jax.ops.segment_max(score, dst, num_segments=N)
        m = jnp.where(jnp.isfinite(m), m, 0.0)
        e = jnp.exp(score - m[dst])
        denom = jax.ops.segment_sum(e, dst, num_segments=N)
        alpha = e / (denom[dst] + 1e-16)
        agg = jax.ops.segment_sum(alpha[:, None] * v[src], dst, num_segments=N)
        f = agg + inp @ Ws + bs
        y = _combine(y, f, jnp.full((1,), dt, jnp.float32))
        ys.append(y)
    return jnp.stack(ys, axis=0)


# speedup vs baseline: 4.4022x; 4.4022x over previous
import functools

import jax
import jax.numpy as jnp
from jax import lax
from jax.experimental import pallas as pl
from jax.experimental.pallas import tpu as pltpu
from jax.experimental.pallas import tpu_sc as plsc

N = 10000
E = 320000
D_IN = 128
H = 256
N_STEPS = 4

NC = 2
NS = 16
NW = NC * NS
CA = 80
CB = 80
NPAD = 10240
EA = E // NW
EB = E // NS

_f32 = jnp.float32
_i32 = jnp.int32

_mesh = plsc.VectorSubcoreMesh(core_axis_name="c", subcore_axis_name="s")
_sc_params = pltpu.CompilerParams(use_tc_tiling_on_sc=False,
                                  needs_layout_passes=False)


@functools.partial(
    pl.kernel,
    out_type=(
        jax.ShapeDtypeStruct((E,), _f32),
        jax.ShapeDtypeStruct((NC, NPAD), _f32),
    ),
    mesh=_mesh,
    scratch_types=[
        pltpu.VMEM((CA, H), _f32),
        pltpu.VMEM((CA, H), _f32),
        pltpu.VMEM((CA,), _i32),
        pltpu.VMEM((CA,), _i32),
        pltpu.VMEM((CA,), _f32),
        pltpu.VMEM((640,), _f32),
        pltpu.VMEM_SHARED((NPAD,), _f32),
        pltpu.SemaphoreType.DMA,
    ],
    compiler_params=_sc_params,
)
def _sc_scores(q_hbm, k_hbm, src_hbm, dst_hbm, e_out, den_out,
               qrows, krows, dstb, srcb, ebuf, zbuf, den_sh, sem):
    c = lax.axis_index("c")
    s = lax.axis_index("s")
    wid = c * NS + s

    def _z(i, _):
        zbuf[pl.ds(i * 16, 16)] = jnp.zeros((16,), _f32)
        return 0
    lax.fori_loop(0, 40, _z, 0)
    pltpu.sync_copy(zbuf, den_sh.at[pl.ds(s * 640, 640)])
    plsc.subcore_barrier()

    ebase = wid * EA
    iota = lax.iota(_i32, 16)

    def _chunk(ci, _):
        off = ebase + ci * CA
        pltpu.sync_copy(dst_hbm.at[pl.ds(off, CA)], dstb)
        pltpu.sync_copy(src_hbm.at[pl.ds(off, CA)], srcb)
        d1 = pltpu.async_copy(q_hbm.at[dstb], qrows, sem)
        d2 = pltpu.async_copy(k_hbm.at[srcb], krows, sem)
        d1.wait()
        d2.wait()

        def _grp(g, _):
            svec = jnp.zeros((16,), _f32)
            for t in range(16):
                e = g * 16 + t
                acc = jnp.zeros((16,), _f32)
                for j in range(H // 16):
                    sl = pl.ds(j * 16, 16)
                    acc = acc + qrows[e, sl] * krows[e, sl]
                svec = jnp.where(iota == t, jnp.sum(acc), svec)
            ebuf[pl.ds(g * 16, 16)] = jnp.exp(svec * 0.0625)
            return 0
        lax.fori_loop(0, CA // 16, _grp, 0)
        pltpu.sync_copy(ebuf, e_out.at[pl.ds(off, CA)])
        pltpu.sync_copy(ebuf, den_sh.at[dstb], add=True)
        return 0
    lax.fori_loop(0, EA // CA, _chunk, 0)

    plsc.subcore_barrier()
    pltpu.sync_copy(den_sh.at[pl.ds(s * 640, 640)],
                    den_out.at[c, pl.ds(s * 640, 640)])


@functools.partial(
    pl.kernel,
    out_type=jax.ShapeDtypeStruct((NC * N, 128), _f32),
    mesh=_mesh,
    scratch_types=[
        pltpu.VMEM((CB, 128), _f32),
        pltpu.VMEM((CB,), _i32),
        pltpu.VMEM((CB,), _i32),
        pltpu.VMEM((CB,), _f32),
        pltpu.VMEM((CB,), _f32),
        pltpu.VMEM((NPAD,), _f32),
        pltpu.VMEM((NPAD,), _f32),
        pltpu.VMEM((16, 128), _f32),
        pltpu.VMEM_SHARED((NPAD, 128), _f32),
        pltpu.SemaphoreType.DMA,
    ],
    compiler_params=_sc_params,
)
def _sc_agg(vh_hbm, src_hbm, dst_hbm, e_hbm, den_hbm, agg_out,
            vrows, dstb, idxb, ebuf, abuf, denv, dtmp, zrows, agg_sh, sem):
    c = lax.axis_index("c")
    s = lax.axis_index("s")

    pltpu.sync_copy(den_hbm.at[0], denv)
    pltpu.sync_copy(den_hbm.at[1], dtmp)

    def _dsum(i, _):
        sl = pl.ds(i * 16, 16)
        denv[sl] = denv[sl] + dtmp[sl] + 1e-30
        return 0
    lax.fori_loop(0, NPAD // 16, _dsum, 0)

    for i in range(16):
        for j in range(8):
            zrows[i, pl.ds(j * 16, 16)] = jnp.zeros((16,), _f32)

    def _zblk(i, _):
        pltpu.sync_copy(zrows, agg_sh.at[pl.ds(s * 640 + i * 16, 16), :])
        return 0
    lax.fori_loop(0, 40, _zblk, 0)
    plsc.subcore_barrier()

    ebase = s * EB
    cbase = c * N

    def _chunk(ci, _):
        off = ebase + ci * CB
        pltpu.sync_copy(dst_hbm.at[pl.ds(off, CB)], dstb)
        pltpu.sync_copy(src_hbm.at[pl.ds(off, CB)], idxb)
        pltpu.sync_copy(e_hbm.at[pl.ds(off, CB)], ebuf)

        def _fix(i, _):
            sl = pl.ds(i * 16, 16)
            idxb[sl] = idxb[sl] + cbase
            return 0
        lax.fori_loop(0, CB // 16, _fix, 0)
        pltpu.async_copy(vh_hbm.at[idxb], vrows, sem).wait()

        def _grp(g, _):
            sl = pl.ds(g * 16, 16)
            den16 = plsc.load_gather(denv, [dstb[sl]])
            abuf[sl] = ebuf[sl] / den16
            return 0
        lax.fori_loop(0, CB // 16, _grp, 0)

        def _scale(e, _):
            a = plsc.load_gather(abuf, [jnp.full((16,), e, _i32)])
            for j in range(128 // 16):
                sl = pl.ds(j * 16, 16)
                vrows[e, sl] = vrows[e, sl] * a
            return 0
        lax.fori_loop(0, CB, _scale, 0)
        pltpu.sync_copy(vrows, agg_sh.at[dstb], add=True)
        return 0
    lax.fori_loop(0, EB // CB, _chunk, 0)

    plsc.subcore_barrier()
    pltpu.sync_copy(agg_sh.at[pl.ds(s * 625, 625), :],
                    agg_out.at[pl.ds(cbase + s * 625, 625), :])


_RB = 1000


def _first_body(x_ref, we_ref, be_ref, w_ref, b_ref,
                h_ref, q_ref, k_ref, vh_ref, s_ref):
    h = jnp.dot(x_ref[...], we_ref[...], preferred_element_type=_f32) + be_ref[...]
    h_ref[...] = h
    o = jnp.dot(h, w_ref[...], preferred_element_type=_f32) + b_ref[...]
    q_ref[...] = o[:, 0:256]
    k_ref[...] = o[:, 256:512]
    vh_ref[0] = o[:, 512:640]
    vh_ref[1] = o[:, 640:768]
    s_ref[...] = o[:, 768:1024]


def _step_body(y_ref, aa_ref, ab_ref, sp_ref, dt_ref, w_ref, b_ref,
               y_out, q_ref, k_ref, vh_ref, s_ref):
    f = jnp.concatenate([aa_ref[...], ab_ref[...]], axis=1) + sp_ref[...]
    y = y_ref[...] + dt_ref[0, 0] * f
    y_out[...] = y
    o = jnp.dot(y, w_ref[...], preferred_element_type=_f32) + b_ref[...]
    q_ref[...] = o[:, 0:256]
    k_ref[...] = o[:, 256:512]
    vh_ref[0] = o[:, 512:640]
    vh_ref[1] = o[:, 640:768]
    s_ref[...] = o[:, 768:1024]


def _final_body(y_ref, aa_ref, ab_ref, sp_ref, dt_ref, y_out):
    f = jnp.concatenate([aa_ref[...], ab_ref[...]], axis=1) + sp_ref[...]
    y_out[...] = y_ref[...] + dt_ref[0, 0] * f


def _qkvs_out():
    return (
        jax.ShapeDtypeStruct((N, H), _f32),
        jax.ShapeDtypeStruct((N, H), _f32),
        jax.ShapeDtypeStruct((2, N, 128), _f32),
        jax.ShapeDtypeStruct((N, H), _f32),
    )


def _qkvs_specs():
    return [
        pl.BlockSpec((_RB, H), lambda i: (i, 0)),
        pl.BlockSpec((_RB, H), lambda i: (i, 0)),
        pl.BlockSpec((2, _RB, 128), lambda i: (0, i, 0)),
        pl.BlockSpec((_RB, H), lambda i: (i, 0)),
    ]


def _tc_first(x, we, be, wcat, bcat):
    return pl.pallas_call(
        _first_body,
        grid=(N // _RB,),
        in_specs=[
            pl.BlockSpec((_RB, D_IN), lambda i: (i, 0)),
            pl.BlockSpec((D_IN, H), lambda i: (0, 0)),
            pl.BlockSpec((1, H), lambda i: (0, 0)),
            pl.BlockSpec((H, 4 * H), lambda i: (0, 0)),
            pl.BlockSpec((1, 4 * H), lambda i: (0, 0)),
        ],
        out_specs=[pl.BlockSpec((_RB, H), lambda i: (i, 0))] + _qkvs_specs(),
        out_shape=(jax.ShapeDtypeStruct((N, H), _f32),) + _qkvs_out(),
    )(x, we, be, wcat, bcat)


def _tc_step(y, aggA, aggB, sp, dtv, wcat, bcat):
    return pl.pallas_call(
        _step_body,
        grid=(N // _RB,),
        in_specs=[
            pl.BlockSpec((_RB, H), lambda i: (i, 0)),
            pl.BlockSpec((_RB, 128), lambda i: (i, 0)),
            pl.BlockSpec((_RB, 128), lambda i: (i, 0)),
            pl.BlockSpec((_RB, H), lambda i: (i, 0)),
            pl.BlockSpec((1, 1), lambda i: (0, 0)),
            pl.BlockSpec((H, 4 * H), lambda i: (0, 0)),
            pl.BlockSpec((1, 4 * H), lambda i: (0, 0)),
        ],
        out_specs=[pl.BlockSpec((_RB, H), lambda i: (i, 0))] + _qkvs_specs(),
        out_shape=(jax.ShapeDtypeStruct((N, H), _f32),) + _qkvs_out(),
    )(y, aggA, aggB, sp, dtv, wcat, bcat)


def _tc_final(y, aggA, aggB, sp, dtv):
    return pl.pallas_call(
        _final_body,
        grid=(N // _RB,),
        in_specs=[
            pl.BlockSpec((_RB, H), lambda i: (i, 0)),
            pl.BlockSpec((_RB, 128), lambda i: (i, 0)),
            pl.BlockSpec((_RB, 128), lambda i: (i, 0)),
            pl.BlockSpec((_RB, H), lambda i: (i, 0)),
            pl.BlockSpec((1, 1), lambda i: (0, 0)),
        ],
        out_specs=pl.BlockSpec((_RB, H), lambda i: (i, 0)),
        out_shape=jax.ShapeDtypeStruct((N, H), _f32),
    )(y, aggA, aggB, sp, dtv)


def kernel(x, edge_index, W_emb, b_emb, Wq, bq, Wk, bk, Wv, bv, Ws, bs):
    src = edge_index[0]
    dst = edge_index[1]
    wcat = jnp.concatenate([Wq[1:], Wk[1:], Wv[1:], Ws[1:]], axis=1)
    w0 = jnp.concatenate([Wq[0], Wk[0], Wv[0], Ws[0]])
    bcat = jnp.concatenate([bq, bk, bv, bs])
    ts = jnp.linspace(0.0, 1.0, N_STEPS)

    b0 = (bcat + ts[0] * w0)[None, :]
    h, q, k, vh, sp = _tc_first(x, W_emb, b_emb[None, :], wcat, b0)
    ys = [h]
    y = h
    for i in range(N_STEPS - 1):
        e, denp = _sc_scores(q, k, src, dst)
        agg = _sc_agg(vh.reshape(2 * N, 128), src, dst, e, denp)
        dtv = (ts[i + 1] - ts[i]).reshape(1, 1)
        if i < N_STEPS - 2:
            bi = (bcat + ts[i + 1] * w0)[None, :]
            y, q, k, vh, sp = _tc_step(y, agg[:N], agg[N:], sp, dtv, wcat, bi)
        else:
            y = _tc_final(y, agg[:N], agg[N:], sp, dtv)
        ys.append(y)
    return jnp.stack(ys, axis=0)
